# Initial kernel scaffold; baseline (speedup 1.0000x reference)
import jax
import jax.numpy as jnp
from jax.experimental import pallas as pl


def kernel(position_embed):
    raise NotImplementedError("write your pallas kernel here")


# speedup vs baseline: 1.3708x; 1.3708x over previous
import functools

import jax
import jax.numpy as jnp
from jax import lax
from jax.experimental import pallas as pl
from jax.experimental.pallas import tpu as pltpu
from jax.experimental.pallas import tpu_sc as plsc

_SEQ_LEN = 8192
_N_DIMS = 128
_NUM_CORES = 2
_NUM_SUBCORES = 16
_NUM_WORKERS = _NUM_CORES * _NUM_SUBCORES
_ROWS_PER_W = _SEQ_LEN // _NUM_WORKERS

_mesh = plsc.VectorSubcoreMesh(core_axis_name="c", subcore_axis_name="s")


@functools.partial(
    pl.kernel,
    mesh=_mesh,
    out_type=jax.ShapeDtypeStruct((_SEQ_LEN, _N_DIMS), jnp.float32),
    scratch_types=[
        pltpu.VMEM((_ROWS_PER_W, _N_DIMS), jnp.float32),
    ],
)
def _position_lookup(table_hbm, out_hbm, buf_v):
    wid = lax.axis_index("s") * _NUM_CORES + lax.axis_index("c")
    base = wid * _ROWS_PER_W
    pltpu.sync_copy(table_hbm.at[pl.ds(base, _ROWS_PER_W)], buf_v)
    pltpu.sync_copy(buf_v, out_hbm.at[pl.ds(base, _ROWS_PER_W)])


def kernel(position_embed):
    return _position_lookup(position_embed)[None]
